# baseline (speedup 1.0000x reference)
import functools

import jax
import jax.numpy as jnp
from jax import lax
from jax.experimental import pallas as pl
from jax.experimental.pallas import tpu as pltpu
from jax.experimental.pallas import tpu_sc as plsc

U = 32
N = 16384
NNZ = 268435
EPS = 1e-10

B = 1024
NNZ_PAD = 294912
CH1 = 18
E1 = CH1 * B
CH4 = 9
E4 = CH4 * B
N_PAD = N + 256
RPW = N_PAD // 16
SUB = 128
TAIL = NNZ % SUB
IDXW = 128
IDR = B // IDXW


def _mesh():
    return plsc.VectorSubcoreMesh(core_axis_name="c", subcore_axis_name="s")


def _sum_scatter_phase(vals_h, cids_h, rids_h, z32_h):
    f32 = jnp.float32

    @functools.partial(
        pl.kernel,
        mesh=_mesh(),
        compiler_params=pltpu.CompilerParams(use_tc_tiling_on_sc=False),
        out_type=jax.ShapeDtypeStruct((2, N_PAD, U), f32),
        scratch_types=[
            pltpu.VMEM((2, 256, U), f32),
            pltpu.VMEM((4, IDXW), jnp.int32),
            pltpu.VMEM_SHARED((N_PAD, U), f32),
            pltpu.SemaphoreType.DMA,
            pltpu.SemaphoreType.DMA,
        ],
    )
    def scat(vals, cids, rids, z32, sum_o, vbuf, ib, sum_sh, lsem, ssem):
        c = lax.axis_index("c")
        s = lax.axis_index("s")
        r0 = s * RPW
        pltpu.sync_copy(z32, sum_sh.at[pl.ds(r0, RPW)])
        plsc.subcore_barrier()

        ebase = s * E1
        nch = E1 // 512

        def chunk(k, carry):
            gb = pl.multiple_of(ebase + k * 512, 512)
            crow = pl.multiple_of(gb // IDXW, 4)

            @pl.when(c == 0)
            def _():
                pltpu.sync_copy(cids.at[pl.ds(crow, 4)], ib)

            @pl.when(c == 1)
            def _():
                pltpu.sync_copy(rids.at[pl.ds(crow, 4)], ib)

            l0 = pltpu.async_copy(vals.at[pl.ds(gb, 256)], vbuf.at[0], lsem)
            sb1 = pl.multiple_of(gb + 256, 256)
            l1 = pltpu.async_copy(vals.at[pl.ds(sb1, 256)], vbuf.at[1], lsem)
            l0.wait()
            s_ = []
            for j in range(2):
                s_.append(pltpu.async_copy(
                    vbuf.at[0].at[pl.ds(j * IDXW, IDXW)],
                    sum_sh.at[ib.at[j]], ssem, add=True))
            l1.wait()
            for j in range(2):
                s_.append(pltpu.async_copy(
                    vbuf.at[1].at[pl.ds(j * IDXW, IDXW)],
                    sum_sh.at[ib.at[2 + j]], ssem, add=True))
            for d in s_:
                d.wait()
            return carry

        lax.fori_loop(0, nch, chunk, 0)
        plsc.subcore_barrier()
        pltpu.sync_copy(sum_sh.at[pl.ds(r0, RPW)], sum_o.at[c, pl.ds(r0, RPW)])

    return scat(vals_h, cids_h, rids_h, z32_h)


def _cnt_scatter_phase(cids_h, rids_h, ones_h, z16_h):
    f32 = jnp.float32

    @functools.partial(
        pl.kernel,
        mesh=_mesh(),
        compiler_params=pltpu.CompilerParams(use_tc_tiling_on_sc=False),
        out_type=jax.ShapeDtypeStruct((2, N_PAD, 16), f32),
        scratch_types=[
            pltpu.VMEM((E1 // IDXW, IDXW), jnp.int32),
            pltpu.VMEM((IDXW, 16), f32),
            pltpu.VMEM_SHARED((N_PAD, 16), f32),
            pltpu.SemaphoreType.DMA,
        ],
    )
    def scat(cids, rids, ones_i, z16, cnt_o, ib, ob, cnt_sh, sem):
        c = lax.axis_index("c")
        s = lax.axis_index("s")
        r0 = s * RPW
        pltpu.sync_copy(z16, cnt_sh.at[pl.ds(r0, RPW)])
        pltpu.sync_copy(ones_i, ob)
        irow = s * (E1 // IDXW)

        @pl.when(c == 0)
        def _():
            pltpu.sync_copy(cids.at[pl.ds(irow, E1 // IDXW)], ib)

        @pl.when(c == 1)
        def _():
            pltpu.sync_copy(rids.at[pl.ds(irow, E1 // IDXW)], ib)

        plsc.subcore_barrier()

        def chunk(k, carry):
            ds_ = []
            for j in range(8):
                ds_.append(pltpu.async_copy(
                    ob, cnt_sh.at[ib.at[k * 8 + j]], sem, add=True))
            for d in ds_:
                d.wait()
            return carry

        lax.fori_loop(0, E1 // IDXW // 8, chunk, 0)
        plsc.subcore_barrier()
        pltpu.sync_copy(cnt_sh.at[pl.ds(r0, RPW)], cnt_o.at[c, pl.ds(r0, RPW)])

    return scat(cids_h, rids_h, ones_h, z16_h)


def _total_phase(sums):
    blocks = 8
    rb = N_PAD // blocks

    def body(cs_ref, o_ref):
        @pl.when(pl.program_id(0) == 0)
        def _():
            o_ref[...] = jnp.zeros_like(o_ref)

        o_ref[...] += jnp.sum(cs_ref[...], axis=(0, 1)).reshape(1, U)

    return pl.pallas_call(
        body,
        grid=(blocks,),
        in_specs=[pl.BlockSpec((1, rb, U), lambda i: (0, i, 0))],
        out_specs=pl.BlockSpec((1, U), lambda i: (0, 0)),
        out_shape=jax.ShapeDtypeStruct((1, U), jnp.float32),
    )(sums)


def _table_phase(sums, cnts, total, th_row, th_col, th_all):
    blocks = 8
    rb = N_PAD // blocks
    f32 = jnp.float32

    def body(sum_ref, cnt_ref, tot_ref, tr_ref, tc_ref, ta_ref,
             trow_o, tcol_o):
        cs = sum_ref[0]
        rs = sum_ref[1]
        ccnt = cnt_ref[0, :, 0:1]
        rcnt = cnt_ref[1, :, 0:1]
        vals_all = jnp.dot(tot_ref[...] * (1.0 / NNZ), ta_ref[...],
                           preferred_element_type=f32)
        trow_o[...] = jnp.dot(cs / (ccnt + EPS), tr_ref[...],
                              preferred_element_type=f32)
        tcol_o[...] = jnp.dot(rs / (rcnt + EPS), tc_ref[...],
                              preferred_element_type=f32) + vals_all

    grid_spec = pl.GridSpec(
        grid=(blocks,),
        in_specs=[
            pl.BlockSpec((2, rb, U), lambda i: (0, i, 0)),
            pl.BlockSpec((2, rb, 16), lambda i: (0, i, 0)),
            pl.BlockSpec((1, U), lambda i: (0, 0)),
            pl.BlockSpec((U, U), lambda i: (0, 0)),
            pl.BlockSpec((U, U), lambda i: (0, 0)),
            pl.BlockSpec((U, U), lambda i: (0, 0)),
        ],
        out_specs=[
            pl.BlockSpec((rb, U), lambda i: (i, 0)),
            pl.BlockSpec((rb, U), lambda i: (i, 0)),
        ],
    )
    return pl.pallas_call(
        body,
        grid_spec=grid_spec,
        out_shape=[
            jax.ShapeDtypeStruct((N_PAD, U), f32),
            jax.ShapeDtypeStruct((N_PAD, U), f32),
        ],
    )(sums, cnts, total, th_row, th_col, th_all)


def _matmul_phase(vals_h, th_sel):
    rb = 8192
    blocks = NNZ_PAD // rb

    def body(v_ref, t_ref, o_ref):
        o_ref[...] = jnp.dot(v_ref[...], t_ref[...],
                             preferred_element_type=jnp.float32)

    return pl.pallas_call(
        body,
        grid=(blocks,),
        in_specs=[
            pl.BlockSpec((rb, U), lambda i: (i, 0)),
            pl.BlockSpec((U, U), lambda i: (0, 0)),
        ],
        out_specs=pl.BlockSpec((rb, U), lambda i: (i, 0)),
        out_shape=jax.ShapeDtypeStruct((NNZ_PAD, U), jnp.float32),
    )(vals_h, th_sel)


def _gather_phase(s_h, cids_h, rids_h, trow_h, tcol_h):
    f32 = jnp.float32
    idrows = E4 // IDXW
    nsub = E4 // SUB
    ipr = SUB // IDXW

    @functools.partial(
        pl.kernel,
        mesh=_mesh(),
        compiler_params=pltpu.CompilerParams(use_tc_tiling_on_sc=False),
        out_type=jax.ShapeDtypeStruct((NNZ, U), f32),
        scratch_types=[
            pltpu.VMEM((2, SUB, U), f32),
            pltpu.VMEM((2, SUB, U), f32),
            pltpu.VMEM((2, SUB, U), f32),
            pltpu.VMEM((2, SUB, U), f32),
            pltpu.VMEM((idrows, IDXW), jnp.int32),
            pltpu.VMEM((idrows, IDXW), jnp.int32),
            pltpu.SemaphoreType.DMA,
            pltpu.SemaphoreType.DMA,
        ],
    )
    def gath(sv, cids, rids, trow, tcol, out_h,
             sbuf, g1, g2, obuf, cb, rb, lsem, ssem):
        c = lax.axis_index("c")
        s = lax.axis_index("s")
        wid = s * 2 + c
        ebase = wid * E4
        irow = wid * idrows
        pltpu.sync_copy(cids.at[pl.ds(irow, idrows)], cb)
        pltpu.sync_copy(rids.at[pl.ds(irow, idrows)], rb)

        def ld_descs(i, p):
            sb = pl.multiple_of(ebase + i * SUB, SUB)
            return [
                (sv.at[pl.ds(sb, SUB)], sbuf.at[p]),
                (trow.at[cb.at[i]], g1.at[p]),
                (tcol.at[rb.at[i]], g2.at[p]),
            ]

        def fire_loads(i, p):
            for src, dst in ld_descs(i, p):
                pltpu.async_copy(src, dst, lsem)

        def drain_loads(i, p):
            for src, dst in ld_descs(i, p):
                pltpu.make_async_copy(src, dst, lsem).wait()

        def compute(i, p):
            sp, gp1, gp2, op_ = sbuf.at[p], g1.at[p], g2.at[p], obuf.at[p]

            def rowbody(ri, rc):
                for rr in range(4):
                    r = ri * 4 + rr
                    for hh in (0, 16):
                        x = (sp[r, pl.ds(hh, 16)] + gp1[r, pl.ds(hh, 16)]
                             + gp2[r, pl.ds(hh, 16)])
                        op_[r, pl.ds(hh, 16)] = jnp.maximum(x, 0.0)
                return rc

            lax.fori_loop(0, SUB // 4, rowbody, 0)

        def store_each(i, p, go):
            sb = pl.multiple_of(ebase + i * SUB, SUB)
            full = sb + SUB <= NNZ

            @pl.when(full)
            def _():
                go(obuf.at[p], out_h.at[pl.ds(sb, SUB)])

            @pl.when(jnp.logical_and(sb < NNZ, jnp.logical_not(full)))
            def _():
                go(obuf.at[p].at[pl.ds(0, TAIL)], out_h.at[pl.ds(sb, TAIL)])

        def fire_store(i, p):
            store_each(i, p, lambda s_, d_: pltpu.async_copy(s_, d_, ssem))

        def drain_store(i, p):
            store_each(i, p,
                       lambda s_, d_: pltpu.make_async_copy(s_, d_, ssem).wait())

        fire_loads(0, 0)

        def body(k, carry):
            i = k * 2
            for p in (0, 1):
                nxt = i + p + 1

                @pl.when(nxt < nsub)
                def _():
                    fire_loads(nxt, (p + 1) % 2)

                drain_loads(i + p, p)

                @pl.when(k > 0)
                def _():
                    drain_store(i + p - 2, p)

                compute(i + p, p)
                fire_store(i + p, p)
            return carry

        lax.fori_loop(0, nsub // 2, body, 0)
        drain_store(nsub - 2, 0)
        drain_store(nsub - 1, 1)

    return gath(s_h, cids_h, rids_h, trow_h, tcol_h)


def kernel(values, indices, theta_sc_sel, theta_sc_row, theta_sc_col,
           theta_sc_all):
    vals2 = values.reshape(-1, U).astype(jnp.float32)
    idx = indices.astype(jnp.int32)
    pad = NNZ_PAD - NNZ
    vals_p = jnp.pad(vals2, ((0, pad), (0, 0)))
    cids = jnp.pad(idx[:, 1], (0, pad), constant_values=N)
    rids = jnp.pad(idx[:, 0], (0, pad), constant_values=N)
    cids2 = cids.reshape(NNZ_PAD // IDXW, IDXW)
    rids2 = rids.reshape(NNZ_PAD // IDXW, IDXW)
    ones_h = jnp.ones((IDXW, 16), jnp.float32)
    z32 = jnp.zeros((RPW, U), jnp.float32)
    z16 = jnp.zeros((RPW, 16), jnp.float32)

    sums = _sum_scatter_phase(vals_p, cids2, rids2, z32)
    cnts = _cnt_scatter_phase(cids2, rids2, ones_h, z16)
    total = _total_phase(sums)
    trow, tcol = _table_phase(sums, cnts, total,
                              theta_sc_row, theta_sc_col, theta_sc_all)
    s = _matmul_phase(vals_p, theta_sc_sel)
    return _gather_phase(s, cids2, rids2, trow, tcol)


# speedup vs baseline: 1.1050x; 1.1050x over previous
import functools

import jax
import jax.numpy as jnp
from jax import lax
from jax.experimental import pallas as pl
from jax.experimental.pallas import tpu as pltpu
from jax.experimental.pallas import tpu_sc as plsc

U = 32
N = 16384
NNZ = 268435
EPS = 1e-10

NNZ_PAD = 294912
E1 = NNZ_PAD // 16
E4 = NNZ_PAD // 32
N_PAD = N + 256
RPW = N_PAD // 16
SUB = 256
TAIL = NNZ % SUB
IDXW = 128


def _mesh():
    return plsc.VectorSubcoreMesh(core_axis_name="c", subcore_axis_name="s")


def _sum_scatter_phase(vals_h, cids_h, rids_h, z32_h):
    f32 = jnp.float32

    @functools.partial(
        pl.kernel,
        mesh=_mesh(),
        compiler_params=pltpu.CompilerParams(use_tc_tiling_on_sc=False),
        out_type=jax.ShapeDtypeStruct((2, N_PAD, U), f32),
        scratch_types=[
            pltpu.VMEM((2, SUB, U), f32),
            pltpu.VMEM((4, IDXW), jnp.int32),
            pltpu.VMEM_SHARED((N_PAD, U), f32),
            pltpu.SemaphoreType.DMA,
        ],
    )
    def scat(vals, cids, rids, z32, sum_o, vbuf, ib, sum_sh, ssem):
        c = lax.axis_index("c")
        s = lax.axis_index("s")
        r0 = s * RPW
        pltpu.sync_copy(z32, sum_sh.at[pl.ds(r0, RPW)])
        plsc.subcore_barrier()

        ebase = s * E1
        nch = E1 // 512

        def chunk(k, carry):
            gb = pl.multiple_of(ebase + k * 512, 512)
            crow = pl.multiple_of(gb // IDXW, 4)

            @pl.when(c == 0)
            def _():
                pltpu.sync_copy(cids.at[pl.ds(crow, 4)], ib)

            @pl.when(c == 1)
            def _():
                pltpu.sync_copy(rids.at[pl.ds(crow, 4)], ib)

            for h in range(2):
                sb = pl.multiple_of(gb + h * SUB, SUB)

                @pl.when(sb + SUB <= NNZ)
                def _():
                    pltpu.sync_copy(vals.at[pl.ds(sb, SUB)], vbuf.at[h])

                @pl.when(jnp.logical_and(sb < NNZ, sb + SUB > NNZ))
                def _():
                    pltpu.sync_copy(vals.at[pl.ds(sb, TAIL)],
                                    vbuf.at[h].at[pl.ds(0, TAIL)])

            s_ = []
            for h in range(2):
                for j in range(2):
                    s_.append(pltpu.async_copy(
                        vbuf.at[h].at[pl.ds(j * IDXW, IDXW)],
                        sum_sh.at[ib.at[h * 2 + j]], ssem, add=True))
            for d in s_:
                d.wait()
            return carry

        lax.fori_loop(0, nch, chunk, 0)
        plsc.subcore_barrier()
        pltpu.sync_copy(sum_sh.at[pl.ds(r0, RPW)], sum_o.at[c, pl.ds(r0, RPW)])

    return scat(vals_h, cids_h, rids_h, z32_h)


def _cnt_scatter_phase(cids_h, rids_h, ones_h, z16_h):
    f32 = jnp.float32

    @functools.partial(
        pl.kernel,
        mesh=_mesh(),
        compiler_params=pltpu.CompilerParams(use_tc_tiling_on_sc=False),
        out_type=jax.ShapeDtypeStruct((2, N_PAD, 16), f32),
        scratch_types=[
            pltpu.VMEM((E1 // IDXW, IDXW), jnp.int32),
            pltpu.VMEM((IDXW, 16), f32),
            pltpu.VMEM_SHARED((N_PAD, 16), f32),
            pltpu.SemaphoreType.DMA,
        ],
    )
    def scat(cids, rids, ones_i, z16, cnt_o, ib, ob, cnt_sh, sem):
        c = lax.axis_index("c")
        s = lax.axis_index("s")
        r0 = s * RPW
        pltpu.sync_copy(z16, cnt_sh.at[pl.ds(r0, RPW)])
        pltpu.sync_copy(ones_i, ob)
        irow = s * (E1 // IDXW)

        @pl.when(c == 0)
        def _():
            pltpu.sync_copy(cids.at[pl.ds(irow, E1 // IDXW)], ib)

        @pl.when(c == 1)
        def _():
            pltpu.sync_copy(rids.at[pl.ds(irow, E1 // IDXW)], ib)

        plsc.subcore_barrier()

        def chunk(k, carry):
            ds_ = []
            for j in range(8):
                ds_.append(pltpu.async_copy(
                    ob, cnt_sh.at[ib.at[k * 8 + j]], sem, add=True))
            for d in ds_:
                d.wait()
            return carry

        lax.fori_loop(0, E1 // IDXW // 8, chunk, 0)
        plsc.subcore_barrier()
        pltpu.sync_copy(cnt_sh.at[pl.ds(r0, RPW)], cnt_o.at[c, pl.ds(r0, RPW)])

    return scat(cids_h, rids_h, ones_h, z16_h)


def _total_phase(sums):
    blocks = 8
    rb = N_PAD // blocks

    def body(cs_ref, o_ref):
        i = pl.program_id(0)

        @pl.when(i == 0)
        def _():
            o_ref[...] = jnp.zeros_like(o_ref)

        row = lax.broadcasted_iota(jnp.int32, (rb, 1), 0) + i * rb
        x = jnp.where(row < N, cs_ref[0], 0.0)
        o_ref[...] += jnp.sum(x, axis=0).reshape(1, U)

    return pl.pallas_call(
        body,
        grid=(blocks,),
        in_specs=[pl.BlockSpec((1, rb, U), lambda i: (0, i, 0))],
        out_specs=pl.BlockSpec((1, U), lambda i: (0, 0)),
        out_shape=jax.ShapeDtypeStruct((1, U), jnp.float32),
    )(sums)


def _table_phase(sums, cnts, total, th_row, th_col, th_all):
    blocks = 8
    rb = N_PAD // blocks
    f32 = jnp.float32

    def body(sum_ref, cnt_ref, tot_ref, tr_ref, tc_ref, ta_ref,
             trow_o, tcol_o):
        cs = sum_ref[0]
        rs = sum_ref[1]
        ccnt = cnt_ref[0, :, 0:1]
        rcnt = cnt_ref[1, :, 0:1]
        vals_all = jnp.dot(tot_ref[...] * (1.0 / NNZ), ta_ref[...],
                           preferred_element_type=f32)
        trow_o[...] = jnp.dot(cs / (ccnt + EPS), tr_ref[...],
                              preferred_element_type=f32)
        tcol_o[...] = jnp.dot(rs / (rcnt + EPS), tc_ref[...],
                              preferred_element_type=f32) + vals_all

    grid_spec = pl.GridSpec(
        grid=(blocks,),
        in_specs=[
            pl.BlockSpec((2, rb, U), lambda i: (0, i, 0)),
            pl.BlockSpec((2, rb, 16), lambda i: (0, i, 0)),
            pl.BlockSpec((1, U), lambda i: (0, 0)),
            pl.BlockSpec((U, U), lambda i: (0, 0)),
            pl.BlockSpec((U, U), lambda i: (0, 0)),
            pl.BlockSpec((U, U), lambda i: (0, 0)),
        ],
        out_specs=[
            pl.BlockSpec((rb, U), lambda i: (i, 0)),
            pl.BlockSpec((rb, U), lambda i: (i, 0)),
        ],
    )
    return pl.pallas_call(
        body,
        grid_spec=grid_spec,
        out_shape=[
            jax.ShapeDtypeStruct((N_PAD, U), f32),
            jax.ShapeDtypeStruct((N_PAD, U), f32),
        ],
    )(sums, cnts, total, th_row, th_col, th_all)


def _matmul_phase(vals_h, th_sel):
    rb = 8192
    blocks = pl.cdiv(NNZ, rb)

    def body(v_ref, t_ref, o_ref):
        o_ref[...] = jnp.dot(v_ref[...], t_ref[...],
                             preferred_element_type=jnp.float32)

    return pl.pallas_call(
        body,
        grid=(blocks,),
        in_specs=[
            pl.BlockSpec((rb, U), lambda i: (i, 0)),
            pl.BlockSpec((U, U), lambda i: (0, 0)),
        ],
        out_specs=pl.BlockSpec((rb, U), lambda i: (i, 0)),
        out_shape=jax.ShapeDtypeStruct((NNZ, U), jnp.float32),
    )(vals_h, th_sel)


def _gather_phase(s_h, cids_h, rids_h, trow_h, tcol_h):
    f32 = jnp.float32
    idrows = E4 // IDXW
    nsub = E4 // SUB
    ipr = SUB // IDXW

    @functools.partial(
        pl.kernel,
        mesh=_mesh(),
        compiler_params=pltpu.CompilerParams(use_tc_tiling_on_sc=False),
        out_type=jax.ShapeDtypeStruct((NNZ, U), f32),
        scratch_types=[
            pltpu.VMEM((SUB, U), f32),
            pltpu.VMEM((SUB, U), f32),
            pltpu.VMEM((SUB, U), f32),
            pltpu.VMEM((idrows, IDXW), jnp.int32),
            pltpu.VMEM((idrows, IDXW), jnp.int32),
            pltpu.SemaphoreType.DMA,
        ],
    )
    def gath(sv, cids, rids, trow, tcol, out_h, sbuf, g1, g2, cb, rb, sem):
        c = lax.axis_index("c")
        s = lax.axis_index("s")
        wid = s * 2 + c
        ebase = wid * E4
        irow = wid * idrows
        pltpu.sync_copy(cids.at[pl.ds(irow, idrows)], cb)
        pltpu.sync_copy(rids.at[pl.ds(irow, idrows)], rb)

        def chunk(k, carry):
            sb = pl.multiple_of(ebase + k * SUB, SUB)
            full = sb + SUB <= NNZ
            part = jnp.logical_and(sb < NNZ, jnp.logical_not(full))

            @pl.when(full)
            def _():
                pltpu.async_copy(sv.at[pl.ds(sb, SUB)], sbuf, sem)

            @pl.when(part)
            def _():
                pltpu.async_copy(sv.at[pl.ds(sb, TAIL)],
                                 sbuf.at[pl.ds(0, TAIL)], sem)

            gds = []
            for j in range(ipr):
                dst = pl.ds(j * IDXW, IDXW)
                gds.append(pltpu.async_copy(trow.at[cb.at[k * ipr + j]],
                                            g1.at[dst], sem))
                gds.append(pltpu.async_copy(tcol.at[rb.at[k * ipr + j]],
                                            g2.at[dst], sem))

            @pl.when(full)
            def _():
                pltpu.make_async_copy(sv.at[pl.ds(sb, SUB)], sbuf, sem).wait()

            @pl.when(part)
            def _():
                pltpu.make_async_copy(sv.at[pl.ds(sb, TAIL)],
                                      sbuf.at[pl.ds(0, TAIL)], sem).wait()

            for d in gds:
                d.wait()

            def rowbody(ri, rc):
                for rr in range(4):
                    r = ri * 4 + rr
                    for hh in (0, 16):
                        x = (sbuf[r, pl.ds(hh, 16)] + g1[r, pl.ds(hh, 16)]
                             + g2[r, pl.ds(hh, 16)])
                        sbuf[r, pl.ds(hh, 16)] = jnp.maximum(x, 0.0)
                return rc

            lax.fori_loop(0, SUB // 4, rowbody, 0)

            @pl.when(full)
            def _():
                pltpu.sync_copy(sbuf, out_h.at[pl.ds(sb, SUB)])

            @pl.when(part)
            def _():
                pltpu.sync_copy(sbuf.at[pl.ds(0, TAIL)],
                                out_h.at[pl.ds(NNZ - TAIL, TAIL)])

            return carry

        lax.fori_loop(0, nsub, chunk, 0)

    return gath(s_h, cids_h, rids_h, trow_h, tcol_h)


def kernel(values, indices, theta_sc_sel, theta_sc_row, theta_sc_col,
           theta_sc_all):
    vals2 = values.reshape(-1, U).astype(jnp.float32)
    idx = indices.astype(jnp.int32)
    pad = NNZ_PAD - NNZ
    cids = jnp.pad(idx[:, 1], (0, pad), constant_values=N)
    rids = jnp.pad(idx[:, 0], (0, pad), constant_values=N)
    cids2 = cids.reshape(NNZ_PAD // IDXW, IDXW)
    rids2 = rids.reshape(NNZ_PAD // IDXW, IDXW)
    ones_h = jnp.ones((IDXW, 16), jnp.float32)
    z32 = jnp.zeros((RPW, U), jnp.float32)
    z16 = jnp.zeros((RPW, 16), jnp.float32)

    sums = _sum_scatter_phase(vals2, cids2, rids2, z32)
    cnts = _cnt_scatter_phase(cids2, rids2, ones_h, z16)
    total = _total_phase(sums)
    trow, tcol = _table_phase(sums, cnts, total,
                              theta_sc_row, theta_sc_col, theta_sc_all)
    s = _matmul_phase(vals2, theta_sc_sel)
    return _gather_phase(s, cids2, rids2, trow, tcol)
